# baseline (speedup 1.0000x reference)
import functools

import jax
import jax.numpy as jnp
from jax import lax
from jax.experimental import pallas as pl
from jax.experimental.pallas import tpu as pltpu
from jax.experimental.pallas import tpu_sc as plsc

_NW = 32
_K = 4


@functools.partial(jax.jit, static_argnames=("n", "q"))
def _sc_gather(flat_ids, W, n, q):
    per_w = n // _NW
    steps = per_w // q
    assert steps % _K == 0 and steps >= 2 * _K

    mesh = plsc.VectorSubcoreMesh(core_axis_name="c", subcore_axis_name="s")

    scratch = []
    for _ in range(_K):
        scratch.append(pltpu.VMEM((4 * q,), jnp.int32))
        scratch.append(pltpu.VMEM((4 * q, 2), jnp.float32))
        scratch.append(pltpu.SemaphoreType.DMA)

    @functools.partial(
        pl.kernel,
        out_type=jax.ShapeDtypeStruct((n, 2), jnp.float32),
        mesh=mesh,
        scratch_types=scratch,
        compiler_params=pltpu.CompilerParams(
            use_tc_tiling_on_sc=False, needs_layout_passes=False
        ),
    )
    def body(ids_hbm, tab_hbm, out_hbm, *bufs):
        wid = lax.axis_index("s") * 2 + lax.axis_index("c")
        base = wid * per_w
        sets = tuple(tuple(bufs[3 * b : 3 * b + 3]) for b in range(_K))

        def stage_and_fire(g, idx_v, rows_v, sem):
            pltpu.sync_copy(ids_hbm.at[pl.ds(base + g * q, q)], idx_v.at[pl.ds(0, q)])
            pltpu.async_copy(tab_hbm.at[idx_v], rows_v, sem)

        for b in range(_K):
            stage_and_fire(b, *sets[b])

        def round_(k, carry):
            for b in range(_K):
                idx_v, rows_v, sem = sets[b]
                g = _K * k + b
                pltpu.make_async_copy(tab_hbm.at[idx_v], rows_v, sem).wait()
                pltpu.sync_copy(rows_v, out_hbm.at[pl.ds(base + g * q, 4 * q)])

                @pl.when(g + _K < steps)
                def _():
                    stage_and_fire(g + _K, idx_v, rows_v, sem)

            return carry

        lax.fori_loop(0, steps // _K, round_, 0)

    return body(flat_ids, W)


def kernel(token_ids, W):
    b, h = token_ids.shape
    n = b * h
    flat = token_ids.reshape(n).astype(jnp.int32)
    out = _sc_gather(flat, W, n, 512)
    return out.reshape(b, h, 2)


# speedup vs baseline: 2.5782x; 2.5782x over previous
import functools

import jax
import jax.numpy as jnp
from jax import lax
from jax.experimental import pallas as pl
from jax.experimental.pallas import tpu as pltpu
from jax.experimental.pallas import tpu_sc as plsc

_NW = 32
_L = 16
_BT = 128
_HT = 8


@functools.partial(jax.jit, static_argnames=("nb", "nh"))
def _sc_gather(ids_phys, table8, nb, nh):
    n_tiles = (nb // _BT) * (nh // _HT)
    per_w = n_tiles // _NW
    assert per_w % 2 == 0
    bt_tiles = nb // _BT
    row_elems = nb * 2

    mesh = plsc.VectorSubcoreMesh(core_axis_name="c", subcore_axis_name="s")

    scratch = []
    for _ in range(2):
        scratch.append(pltpu.VMEM((_BT * _HT,), jnp.int32))
        scratch.append(pltpu.VMEM((_BT * _HT,), jnp.int32))
        scratch.append(pltpu.VMEM((_BT * _HT, 8), jnp.float32))
        scratch.append(pltpu.VMEM((_HT, 2 * _BT), jnp.float32))
        scratch.append(pltpu.SemaphoreType.DMA)

    @functools.partial(
        pl.kernel,
        out_type=jax.ShapeDtypeStruct((nh, row_elems), jnp.float32),
        mesh=mesh,
        scratch_types=scratch,
        compiler_params=pltpu.CompilerParams(
            use_tc_tiling_on_sc=False, needs_layout_passes=False
        ),
    )
    def body(ids_hbm, tab_hbm, out_hbm, *bufs):
        wid = lax.axis_index("s") * 2 + lax.axis_index("c")
        t_base = wid * per_w
        sets = tuple(tuple(bufs[5 * b : 5 * b + 5]) for b in range(2))
        iota = lax.iota(jnp.int32, _L)
        tile_n = _BT * _HT
        groups = tile_n // _L

        def stage_and_fire(t, idx_v, line_v, rows_v, sem):
            pltpu.sync_copy(ids_hbm.at[pl.ds(t * tile_n, tile_n)], idx_v)

            def lines(g, carry):
                v = idx_v[pl.ds(g * _L, _L)]
                line_v[pl.ds(g * _L, _L)] = lax.shift_right_logical(v, 2)
                return carry

            lax.fori_loop(0, groups, lines, 0)
            pltpu.async_copy(tab_hbm.at[line_v], rows_v, sem)

        for b in range(2):
            idx_v, line_v, rows_v, comp_v, sem = sets[b]
            stage_and_fire(t_base + b, idx_v, line_v, rows_v, sem)

        def pair(k, carry):
            for b in range(2):
                idx_v, line_v, rows_v, comp_v, sem = sets[b]
                t = t_base + 2 * k + b
                pltpu.make_async_copy(tab_hbm.at[line_v], rows_v, sem).wait()

                for hh in range(_HT):

                    def extract(g, carry2, hh=hh):
                        p = hh * _BT + g * _L
                        v = idx_v[pl.ds(p, _L)]
                        col = lax.shift_left(v & 3, 1)
                        r16 = iota + p
                        x = plsc.load_gather(rows_v, [r16, col])
                        y = plsc.load_gather(rows_v, [r16, col + 1])
                        comp_v[hh, pl.ds(g * _L, _L)] = x
                        comp_v[hh, pl.ds(_BT + g * _L, _L)] = y
                        return carry2

                    lax.fori_loop(0, _BT // _L, extract, 0)

                @pl.when(2 * k + b + 2 < per_w)
                def _():
                    stage_and_fire(t + 2, idx_v, line_v, rows_v, sem)

                ht = t // bt_tiles
                bt = t - ht * bt_tiles
                pltpu.sync_copy(
                    comp_v,
                    out_hbm.at[pl.ds(ht * _HT, _HT), pl.ds(bt * 2 * _BT, 2 * _BT)],
                )
            return carry

        lax.fori_loop(0, per_w // 2, pair, 0)

    return body(ids_phys, table8)


def kernel(token_ids, W):
    nb, nh = token_ids.shape
    bt_tiles = nb // _BT
    ht_tiles = nh // _HT
    ids_phys = (
        token_ids.astype(jnp.int32)
        .reshape(bt_tiles, _BT, ht_tiles, _HT)
        .transpose(2, 0, 3, 1)
        .reshape(nb * nh)
    )
    table8 = W.reshape(W.shape[0] // 4, 8)
    out2d = _sc_gather(ids_phys, table8, nb, nh)
    return (
        out2d.reshape(nh, bt_tiles, 2, _BT)
        .transpose(1, 3, 0, 2)
        .reshape(nb, nh, 2)
    )


# speedup vs baseline: 12.5787x; 4.8789x over previous
import functools

import jax
import jax.numpy as jnp
from jax import lax
from jax.experimental import pallas as pl
from jax.experimental.pallas import tpu as pltpu
from jax.experimental.pallas import tpu_sc as plsc

_NW = 32
_L = 16
_BT = 128
_HT = 8


@functools.partial(jax.jit, static_argnames=("nb", "nh"))
def _sc_gather(ids_phys, w_phys, nb, nh):
    n_tiles = (nb // _BT) * (nh // _HT)
    per_w = n_tiles // _NW
    assert per_w % 2 == 0
    bt_tiles = nb // _BT
    row_elems = nb * 2
    tile_n = _BT * _HT

    mesh = plsc.VectorSubcoreMesh(core_axis_name="c", subcore_axis_name="s")

    scratch = []
    for _ in range(2):
        scratch.append(pltpu.VMEM((tile_n,), jnp.int32))
        scratch.append(pltpu.VMEM((2 * tile_n,), jnp.int32))
        scratch.append(pltpu.VMEM((2 * tile_n,), jnp.float32))
        scratch.append(pltpu.SemaphoreType.DMA)
        scratch.append(pltpu.SemaphoreType.DMA)

    @functools.partial(
        pl.kernel,
        out_type=jax.ShapeDtypeStruct((nh * row_elems,), jnp.float32),
        mesh=mesh,
        scratch_types=scratch,
        compiler_params=pltpu.CompilerParams(
            use_tc_tiling_on_sc=False, needs_layout_passes=False
        ),
    )
    def body(ids_hbm, tab_hbm, out_hbm, *bufs):
        wid = lax.axis_index("s") * 2 + lax.axis_index("c")
        t_base = wid * per_w
        sets = tuple(tuple(bufs[5 * b : 5 * b + 5]) for b in range(2))

        def stage_and_pos(t, idx_v, pos_v):
            pltpu.sync_copy(ids_hbm.at[pl.ds(t * tile_n, tile_n)], idx_v)
            for hh in range(_HT):

                def positions(g, carry, hh=hh):
                    v = idx_v[pl.ds(hh * _BT + g * _L, _L)]
                    px = lax.shift_left(lax.shift_right_logical(v, 7), 8) + (
                        v & (_BT - 1)
                    )
                    pos_v[pl.ds(hh * 2 * _BT + g * _L, _L)] = px
                    pos_v[pl.ds(hh * 2 * _BT + _BT + g * _L, _L)] = px + _BT
                    return carry

                lax.fori_loop(0, _BT // _L, positions, 0)

        def wb_slices(t, blk_v):
            ht = t // bt_tiles
            bt = t - ht * bt_tiles
            for hh in range(_HT):
                src = blk_v.at[pl.ds(hh * 2 * _BT, 2 * _BT)]
                dst = out_hbm.at[
                    pl.ds((ht * _HT + hh) * row_elems + bt * 2 * _BT, 2 * _BT)
                ]
                yield src, dst

        for b in range(2):
            idx_v, pos_v, blk_v, sem, wsem = sets[b]
            stage_and_pos(t_base + b, idx_v, pos_v)
            pltpu.async_copy(tab_hbm.at[pos_v], blk_v, sem)

        def pair(k, carry):
            for b in range(2):
                idx_v, pos_v, blk_v, sem, wsem = sets[b]
                t = t_base + 2 * k + b
                pltpu.make_async_copy(tab_hbm.at[pos_v], blk_v, sem).wait()
                for src, dst in wb_slices(t, blk_v):
                    pltpu.async_copy(src, dst, wsem)

                @pl.when(2 * k + b + 2 < per_w)
                def _():
                    stage_and_pos(t + 2, idx_v, pos_v)

                for src, dst in wb_slices(t, blk_v):
                    pltpu.make_async_copy(src, dst, wsem).wait()

                @pl.when(2 * k + b + 2 < per_w)
                def _():
                    pltpu.async_copy(tab_hbm.at[pos_v], blk_v, sem)

            return carry

        lax.fori_loop(0, per_w // 2, pair, 0)

    return body(ids_phys, w_phys)


def kernel(token_ids, W):
    nb, nh = token_ids.shape
    bt_tiles = nb // _BT
    ht_tiles = nh // _HT
    ids_phys = (
        token_ids.astype(jnp.int32)
        .reshape(bt_tiles, _BT, ht_tiles, _HT)
        .transpose(2, 0, 3, 1)
        .reshape(nb * nh)
    )
    v = W.shape[0]
    vp = (v + _BT - 1) // _BT * _BT
    w_phys = (
        jnp.pad(W, ((0, vp - v), (0, 0)))
        .reshape(vp // _BT, _BT, 2)
        .transpose(0, 2, 1)
        .reshape(2 * vp)
    )
    out1d = _sc_gather(ids_phys, w_phys, nb, nh)
    return (
        out1d.reshape(nh, bt_tiles, 2, _BT)
        .transpose(1, 3, 0, 2)
        .reshape(nb, nh, 2)
    )


# speedup vs baseline: 16.0509x; 1.2760x over previous
import functools

import jax
import jax.numpy as jnp
from jax import lax
from jax.experimental import pallas as pl
from jax.experimental.pallas import tpu as pltpu
from jax.experimental.pallas import tpu_sc as plsc

_NW = 32
_NS = 16
_L = 16
_BT = 128
_HT = 8
_VPAD = 65536
_CB = 16


@functools.partial(jax.jit, static_argnames=("nb", "nh", "vp"))
def _sc_gather(ids_phys, w_phys, nb, nh, vp):
    n_tiles = (nb // _BT) * (nh // _HT)
    per_w = n_tiles // _NW
    assert per_w % 2 == 0
    bt_tiles = nb // _BT
    row_elems = nb * 2
    tile_n = _BT * _HT

    n_blocks = vp // _BT
    chunks_per_s = n_blocks // _CB // _NS
    assert n_blocks % (_CB * _NS) == 0
    cf = _CB * 2 * _BT
    cl = _CB * _BT // 4

    mesh = plsc.VectorSubcoreMesh(core_axis_name="c", subcore_axis_name="s")

    scratch = []
    for _ in range(2):
        scratch.append(pltpu.VMEM((cf,), jnp.float32))
        scratch.append(pltpu.VMEM((cl, 8), jnp.float32))
        scratch.append(pltpu.SemaphoreType.DMA)
        scratch.append(pltpu.SemaphoreType.DMA)
    for _ in range(2):
        scratch.append(pltpu.VMEM((tile_n,), jnp.int32))
        scratch.append(pltpu.VMEM((tile_n,), jnp.int32))
        scratch.append(pltpu.VMEM((tile_n, 8), jnp.float32))
        scratch.append(pltpu.VMEM((_HT, 2 * _BT), jnp.float32))
        scratch.append(pltpu.SemaphoreType.DMA)

    @functools.partial(
        pl.kernel,
        out_type=(
            jax.ShapeDtypeStruct((nh, row_elems), jnp.float32),
            jax.ShapeDtypeStruct((vp // 4, 8), jnp.float32),
        ),
        mesh=mesh,
        scratch_types=scratch,
        compiler_params=pltpu.CompilerParams(
            use_tc_tiling_on_sc=False, needs_layout_passes=False
        ),
    )
    def body(ids_hbm, tab_hbm, out_hbm, lines_hbm, *bufs):
        cid = lax.axis_index("c")
        sid = lax.axis_index("s")
        wid = sid * 2 + cid
        t_base = wid * per_w
        psets = tuple(tuple(bufs[4 * b : 4 * b + 4]) for b in range(2))
        gsets = tuple(tuple(bufs[8 + 5 * b : 8 + 5 * b + 5]) for b in range(2))
        iota = lax.iota(jnp.int32, _L)

        def p1_stage(ci, ib_v, isem):
            pltpu.async_copy(
                tab_hbm.at[pl.ds((sid * chunks_per_s + ci) * cf, cf)], ib_v, isem
            )

        def p1_work(ci, ib_v, pr_v, isem, wsem):
            pltpu.make_async_copy(
                tab_hbm.at[pl.ds((sid * chunks_per_s + ci) * cf, cf)], ib_v, isem
            ).wait()

            def inter(g, carry):
                blk = lax.shift_right_logical(g, 3)
                off = (g & 7) * _L
                src = blk * 2 * _BT + off
                x16 = ib_v[pl.ds(src, _L)]
                y16 = ib_v[pl.ds(src + _BT, _L)]
                p = blk * _BT + off + iota
                row = lax.shift_right_logical(p, 2)
                colx = lax.shift_left(p & 3, 1)
                plsc.store_scatter(pr_v, [row, colx], x16)
                plsc.store_scatter(pr_v, [row, colx + 1], y16)
                return carry

            lax.fori_loop(0, _CB * _BT // _L, inter, 0)
            pltpu.async_copy(
                pr_v,
                lines_hbm.at[pl.ds((sid * chunks_per_s + ci) * cl, cl), :],
                wsem,
            )

        def p1_drain(ci, pr_v, wsem):
            pltpu.make_async_copy(
                pr_v,
                lines_hbm.at[pl.ds((sid * chunks_per_s + ci) * cl, cl), :],
                wsem,
            ).wait()

        for b in range(2):
            p1_stage(b, psets[b][0], psets[b][2])

        def p1_pair(k, carry):
            for b in range(2):
                ib_v, pr_v, isem, wsem = psets[b]
                ci = 2 * k + b

                @pl.when(ci >= 2)
                def _():
                    p1_drain(ci - 2, pr_v, wsem)

                p1_work(ci, ib_v, pr_v, isem, wsem)

                @pl.when(ci + 2 < chunks_per_s)
                def _():
                    p1_stage(ci + 2, ib_v, isem)

            return carry

        lax.fori_loop(0, chunks_per_s // 2, p1_pair, 0)
        for b in range(2):
            p1_drain(chunks_per_s - 2 + b, psets[b][1], psets[b][3])
        plsc.subcore_barrier()

        def stage_and_fire(t, idx_v, line_v, rows_v, sem):
            pltpu.sync_copy(ids_hbm.at[pl.ds(t * tile_n, tile_n)], idx_v)

            def lines(g, carry):
                v = idx_v[pl.ds(g * _L, _L)]
                line_v[pl.ds(g * _L, _L)] = lax.shift_right_logical(v, 2)
                return carry

            lax.fori_loop(0, tile_n // _L, lines, 0)
            pltpu.async_copy(lines_hbm.at[line_v], rows_v, sem)

        for b in range(2):
            idx_v, line_v, rows_v, comp_v, sem = gsets[b]
            stage_and_fire(t_base + b, idx_v, line_v, rows_v, sem)

        def pair(k, carry):
            for b in range(2):
                idx_v, line_v, rows_v, comp_v, sem = gsets[b]
                t = t_base + 2 * k + b
                pltpu.make_async_copy(lines_hbm.at[line_v], rows_v, sem).wait()

                for hh in range(_HT):

                    def extract(g, carry2, hh=hh):
                        p = hh * _BT + g * _L
                        v = idx_v[pl.ds(p, _L)]
                        col = lax.shift_left(v & 3, 1)
                        r16 = iota + p
                        x = plsc.load_gather(rows_v, [r16, col])
                        y = plsc.load_gather(rows_v, [r16, col + 1])
                        comp_v[hh, pl.ds(g * _L, _L)] = x
                        comp_v[hh, pl.ds(_BT + g * _L, _L)] = y
                        return carry2

                    lax.fori_loop(0, _BT // _L, extract, 0)

                @pl.when(2 * k + b + 2 < per_w)
                def _():
                    stage_and_fire(t + 2, idx_v, line_v, rows_v, sem)

                ht = t // bt_tiles
                bt = t - ht * bt_tiles
                pltpu.sync_copy(
                    comp_v,
                    out_hbm.at[pl.ds(ht * _HT, _HT), pl.ds(bt * 2 * _BT, 2 * _BT)],
                )
            return carry

        lax.fori_loop(0, per_w // 2, pair, 0)

    return body(ids_phys, w_phys)


def kernel(token_ids, W):
    nb, nh = token_ids.shape
    bt_tiles = nb // _BT
    ht_tiles = nh // _HT
    ids_phys = (
        token_ids.astype(jnp.int32)
        .reshape(bt_tiles, _BT, ht_tiles, _HT)
        .transpose(2, 0, 3, 1)
        .reshape(nb * nh)
    )
    v = W.shape[0]
    vp = (v + _VPAD - 1) // _VPAD * _VPAD
    w_phys = (
        jnp.pad(W, ((0, vp - v), (0, 0)))
        .reshape(vp // _BT, _BT, 2)
        .transpose(0, 2, 1)
        .reshape(2 * vp)
    )
    out2d, _ = _sc_gather(ids_phys, w_phys, nb, nh, vp)
    return (
        out2d.reshape(nh, bt_tiles, 2, _BT)
        .transpose(1, 3, 0, 2)
        .reshape(nb, nh, 2)
    )


# speedup vs baseline: 16.2653x; 1.0134x over previous
import functools

import jax
import jax.numpy as jnp
from jax import lax
from jax.experimental import pallas as pl
from jax.experimental.pallas import tpu as pltpu
from jax.experimental.pallas import tpu_sc as plsc

_NW = 32
_NS = 16
_L = 16
_BT = 128
_HT = 8
_VPAD = 65536
_CB = 16


@functools.partial(jax.jit, static_argnames=("nb", "nh", "vp"))
def _sc_gather(ids_phys, w_phys, nb, nh, vp):
    n_tiles = (nb // _BT) * (nh // _HT)
    per_w = n_tiles // _NW
    assert per_w % 2 == 0
    bt_tiles = nb // _BT
    row_elems = nb * 2
    tile_n = _BT * _HT

    n_blocks = vp // _BT
    chunks_per_s = n_blocks // _CB // _NS
    assert n_blocks % (_CB * _NS) == 0
    cf = _CB * 2 * _BT
    cl = _CB * _BT // 4

    mesh = plsc.VectorSubcoreMesh(core_axis_name="c", subcore_axis_name="s")

    scratch = []
    for _ in range(2):
        scratch.append(pltpu.VMEM((cf,), jnp.float32))
        scratch.append(pltpu.VMEM((cl, 8), jnp.float32))
        scratch.append(pltpu.SemaphoreType.DMA)
        scratch.append(pltpu.SemaphoreType.DMA)
    for _ in range(2):
        scratch.append(pltpu.VMEM((tile_n,), jnp.int32))
        scratch.append(pltpu.VMEM((tile_n,), jnp.int32))
        scratch.append(pltpu.VMEM((tile_n, 8), jnp.float32))
        scratch.append(pltpu.VMEM((_HT, 2 * _BT), jnp.float32))
        scratch.append(pltpu.SemaphoreType.DMA)
        scratch.append(pltpu.SemaphoreType.DMA)

    @functools.partial(
        pl.kernel,
        out_type=(
            jax.ShapeDtypeStruct((nh, row_elems), jnp.float32),
            jax.ShapeDtypeStruct((vp // 4, 8), jnp.float32),
        ),
        mesh=mesh,
        scratch_types=scratch,
        compiler_params=pltpu.CompilerParams(
            use_tc_tiling_on_sc=False, needs_layout_passes=False
        ),
    )
    def body(ids_hbm, tab_hbm, out_hbm, lines_hbm, *bufs):
        cid = lax.axis_index("c")
        sid = lax.axis_index("s")
        wid = sid * 2 + cid
        t_base = wid * per_w
        psets = tuple(tuple(bufs[4 * b : 4 * b + 4]) for b in range(2))
        gsets = tuple(tuple(bufs[8 + 6 * b : 8 + 6 * b + 6]) for b in range(2))
        iota = lax.iota(jnp.int32, _L)

        def p1_stage(ci, ib_v, isem):
            pltpu.async_copy(
                tab_hbm.at[pl.ds((sid * chunks_per_s + ci) * cf, cf)], ib_v, isem
            )

        def p1_work(ci, ib_v, pr_v, isem, wsem):
            pltpu.make_async_copy(
                tab_hbm.at[pl.ds((sid * chunks_per_s + ci) * cf, cf)], ib_v, isem
            ).wait()

            def inter(g, carry):
                blk = lax.shift_right_logical(g, 3)
                off = (g & 7) * _L
                src = blk * 2 * _BT + off
                x16 = ib_v[pl.ds(src, _L)]
                y16 = ib_v[pl.ds(src + _BT, _L)]
                p = blk * _BT + off + iota
                row = lax.shift_right_logical(p, 2)
                colx = lax.shift_left(p & 3, 1)
                plsc.store_scatter(pr_v, [row, colx], x16)
                plsc.store_scatter(pr_v, [row, colx + 1], y16)
                return carry

            lax.fori_loop(0, _CB * _BT // _L, inter, 0)
            pltpu.async_copy(
                pr_v,
                lines_hbm.at[pl.ds((sid * chunks_per_s + ci) * cl, cl), :],
                wsem,
            )

        def p1_drain(ci, pr_v, wsem):
            pltpu.make_async_copy(
                pr_v,
                lines_hbm.at[pl.ds((sid * chunks_per_s + ci) * cl, cl), :],
                wsem,
            ).wait()

        for b in range(2):
            p1_stage(b, psets[b][0], psets[b][2])

        def p1_pair(k, carry):
            for b in range(2):
                ib_v, pr_v, isem, wsem = psets[b]
                ci = 2 * k + b

                @pl.when(ci >= 2)
                def _():
                    p1_drain(ci - 2, pr_v, wsem)

                p1_work(ci, ib_v, pr_v, isem, wsem)

                @pl.when(ci + 2 < chunks_per_s)
                def _():
                    p1_stage(ci + 2, ib_v, isem)

            return carry

        lax.fori_loop(0, chunks_per_s // 2, p1_pair, 0)
        for b in range(2):
            p1_drain(chunks_per_s - 2 + b, psets[b][1], psets[b][3])
        plsc.subcore_barrier()

        def stage_and_fire(t, idx_v, line_v, rows_v, sem):
            pltpu.sync_copy(ids_hbm.at[pl.ds(t * tile_n, tile_n)], idx_v)

            def lines(g, carry):
                v = idx_v[pl.ds(g * _L, _L)]
                line_v[pl.ds(g * _L, _L)] = lax.shift_right_logical(v, 2)
                return carry

            lax.fori_loop(0, tile_n // _L, lines, 0)
            pltpu.async_copy(lines_hbm.at[line_v], rows_v, sem)

        def wb_dst(t):
            ht = t // bt_tiles
            bt = t - ht * bt_tiles
            return out_hbm.at[
                pl.ds(ht * _HT, _HT), pl.ds(bt * 2 * _BT, 2 * _BT)
            ]

        for b in range(2):
            idx_v, line_v, rows_v, comp_v, sem, wsem = gsets[b]
            stage_and_fire(t_base + b, idx_v, line_v, rows_v, sem)

        def pair(k, carry):
            for b in range(2):
                idx_v, line_v, rows_v, comp_v, sem, wsem = gsets[b]
                t = t_base + 2 * k + b
                pltpu.make_async_copy(lines_hbm.at[line_v], rows_v, sem).wait()

                @pl.when(2 * k + b >= 2)
                def _():
                    pltpu.make_async_copy(comp_v, wb_dst(t - 2), wsem).wait()

                for hh in range(_HT):

                    def extract(g, carry2, hh=hh):
                        p = hh * _BT + g * _L
                        v = idx_v[pl.ds(p, _L)]
                        col = lax.shift_left(v & 3, 1)
                        r16 = iota + p
                        x = plsc.load_gather(rows_v, [r16, col])
                        y = plsc.load_gather(rows_v, [r16, col + 1])
                        comp_v[hh, pl.ds(g * _L, _L)] = x
                        comp_v[hh, pl.ds(_BT + g * _L, _L)] = y
                        return carry2

                    lax.fori_loop(0, _BT // _L, extract, 0)

                @pl.when(2 * k + b + 2 < per_w)
                def _():
                    stage_and_fire(t + 2, idx_v, line_v, rows_v, sem)

                pltpu.async_copy(comp_v, wb_dst(t), wsem)
            return carry

        lax.fori_loop(0, per_w // 2, pair, 0)
        for b in range(2):
            idx_v, line_v, rows_v, comp_v, sem, wsem = gsets[b]
            pltpu.make_async_copy(comp_v, wb_dst(t_base + per_w - 2 + b), wsem).wait()

    return body(ids_phys, w_phys)


def kernel(token_ids, W):
    nb, nh = token_ids.shape
    bt_tiles = nb // _BT
    ht_tiles = nh // _HT
    ids_phys = (
        token_ids.astype(jnp.int32)
        .reshape(bt_tiles, _BT, ht_tiles, _HT)
        .transpose(2, 0, 3, 1)
        .reshape(nb * nh)
    )
    v = W.shape[0]
    vp = (v + _VPAD - 1) // _VPAD * _VPAD
    w_phys = (
        jnp.pad(W, ((0, vp - v), (0, 0)))
        .reshape(vp // _BT, _BT, 2)
        .transpose(0, 2, 1)
        .reshape(2 * vp)
    )
    out2d, _ = _sc_gather(ids_phys, w_phys, nb, nh, vp)
    return (
        out2d.reshape(nh, bt_tiles, 2, _BT)
        .transpose(1, 3, 0, 2)
        .reshape(nb, nh, 2)
    )


# speedup vs baseline: 18.4405x; 1.1337x over previous
import functools

import jax
import jax.numpy as jnp
from jax import lax
from jax.experimental import pallas as pl
from jax.experimental.pallas import tpu as pltpu
from jax.experimental.pallas import tpu_sc as plsc

_NW = 32
_NS = 16
_L = 16
_BT = 128
_HT = 8
_VPAD = 65536
_CB = 16


@functools.partial(jax.jit, static_argnames=("nb", "nh", "vp"))
def _sc_gather(ids_phys, w_phys, nb, nh, vp):
    n_tiles = (nb // _BT) * (nh // _HT)
    per_w = n_tiles // _NW
    assert per_w % 2 == 0
    bt_tiles = nb // _BT
    row_elems = nb * 2
    tile_n = _BT * _HT

    n_blocks = vp // _BT
    chunks_per_s = n_blocks // _CB // _NS
    assert n_blocks % (_CB * _NS) == 0
    cf = _CB * 2 * _BT
    cl = _CB * _BT // 4

    mesh = plsc.VectorSubcoreMesh(core_axis_name="c", subcore_axis_name="s")

    scratch = []
    for _ in range(2):
        scratch.append(pltpu.VMEM((cf,), jnp.float32))
        scratch.append(pltpu.VMEM((cl, 8), jnp.float32))
        scratch.append(pltpu.SemaphoreType.DMA)
        scratch.append(pltpu.SemaphoreType.DMA)
    for _ in range(2):
        scratch.append(pltpu.VMEM((tile_n,), jnp.int32))
        scratch.append(pltpu.VMEM((tile_n,), jnp.int32))
        scratch.append(pltpu.VMEM((tile_n, 8), jnp.float32))
        scratch.append(pltpu.VMEM((_HT, 2 * _BT), jnp.float32))
        scratch.append(pltpu.SemaphoreType.DMA)
        scratch.append(pltpu.SemaphoreType.DMA)

    @functools.partial(
        pl.kernel,
        out_type=(
            jax.ShapeDtypeStruct((nh, row_elems), jnp.float32),
            jax.ShapeDtypeStruct((vp // 4, 8), jnp.float32),
        ),
        mesh=mesh,
        scratch_types=scratch,
        compiler_params=pltpu.CompilerParams(
            use_tc_tiling_on_sc=False, needs_layout_passes=False
        ),
    )
    def body(ids_hbm, tab_hbm, out_hbm, lines_hbm, *bufs):
        cid = lax.axis_index("c")
        sid = lax.axis_index("s")
        wid = sid * 2 + cid
        t_base = wid * per_w
        psets = tuple(tuple(bufs[4 * b : 4 * b + 4]) for b in range(2))
        gsets = tuple(tuple(bufs[8 + 6 * b : 8 + 6 * b + 6]) for b in range(2))
        iota = lax.iota(jnp.int32, _L)

        def p1_stage(ci, ib_v, isem):
            pltpu.async_copy(
                tab_hbm.at[pl.ds((sid * chunks_per_s + ci) * cf, cf)], ib_v, isem
            )

        def p1_work(ci, ib_v, pr_v, isem, wsem):
            pltpu.make_async_copy(
                tab_hbm.at[pl.ds((sid * chunks_per_s + ci) * cf, cf)], ib_v, isem
            ).wait()

            def inter(g, carry):
                blk = lax.shift_right_logical(g, 3)
                off = (g & 7) * _L
                src = blk * 2 * _BT + off
                x16 = ib_v[pl.ds(src, _L)]
                y16 = ib_v[pl.ds(src + _BT, _L)]
                p = blk * _BT + off + iota
                row = lax.shift_right_logical(p, 2)
                colx = lax.shift_left(p & 3, 1)
                plsc.store_scatter(pr_v, [row, colx], x16)
                plsc.store_scatter(pr_v, [row, colx + 1], y16)
                return carry

            lax.fori_loop(0, _CB * _BT // _L, inter, 0)
            pltpu.async_copy(
                pr_v,
                lines_hbm.at[pl.ds((sid * chunks_per_s + ci) * cl, cl), :],
                wsem,
            )

        def p1_drain(ci, pr_v, wsem):
            pltpu.make_async_copy(
                pr_v,
                lines_hbm.at[pl.ds((sid * chunks_per_s + ci) * cl, cl), :],
                wsem,
            ).wait()

        for b in range(2):
            p1_stage(b, psets[b][0], psets[b][2])

        def p1_pair(k, carry):
            for b in range(2):
                ib_v, pr_v, isem, wsem = psets[b]
                ci = 2 * k + b

                @pl.when(ci >= 2)
                def _():
                    p1_drain(ci - 2, pr_v, wsem)

                p1_work(ci, ib_v, pr_v, isem, wsem)

                @pl.when(ci + 2 < chunks_per_s)
                def _():
                    p1_stage(ci + 2, ib_v, isem)

            return carry

        lax.fori_loop(0, chunks_per_s // 2, p1_pair, 0)
        for b in range(2):
            p1_drain(chunks_per_s - 2 + b, psets[b][1], psets[b][3])
        plsc.subcore_barrier()

        def stage_and_fire(t, idx_v, line_v, rows_v, sem):
            pltpu.sync_copy(ids_hbm.at[pl.ds(t * tile_n, tile_n)], idx_v)

            @plsc.parallel_loop(0, tile_n // _L, unroll=4)
            def lines(g):
                v = idx_v[pl.ds(g * _L, _L)]
                line_v[pl.ds(g * _L, _L)] = lax.shift_right_logical(v, 2)

            pltpu.async_copy(lines_hbm.at[line_v], rows_v, sem)

        def wb_dst(t):
            ht = t // bt_tiles
            bt = t - ht * bt_tiles
            return out_hbm.at[
                pl.ds(ht * _HT, _HT), pl.ds(bt * 2 * _BT, 2 * _BT)
            ]

        for b in range(2):
            idx_v, line_v, rows_v, comp_v, sem, wsem = gsets[b]
            stage_and_fire(t_base + b, idx_v, line_v, rows_v, sem)

        def pair(k, carry):
            for b in range(2):
                idx_v, line_v, rows_v, comp_v, sem, wsem = gsets[b]
                t = t_base + 2 * k + b
                pltpu.make_async_copy(lines_hbm.at[line_v], rows_v, sem).wait()

                @pl.when(2 * k + b >= 2)
                def _():
                    pltpu.make_async_copy(comp_v, wb_dst(t - 2), wsem).wait()

                for hh in range(_HT):

                    @plsc.parallel_loop(0, _BT // _L, unroll=4)
                    def extract(g, hh=hh):
                        p = hh * _BT + g * _L
                        v = idx_v[pl.ds(p, _L)]
                        col = lax.shift_left(v & 3, 1)
                        r16 = iota + p
                        x = plsc.load_gather(rows_v, [r16, col])
                        y = plsc.load_gather(rows_v, [r16, col + 1])
                        comp_v[hh, pl.ds(g * _L, _L)] = x
                        comp_v[hh, pl.ds(_BT + g * _L, _L)] = y

                @pl.when(2 * k + b + 2 < per_w)
                def _():
                    stage_and_fire(t + 2, idx_v, line_v, rows_v, sem)

                pltpu.async_copy(comp_v, wb_dst(t), wsem)
            return carry

        lax.fori_loop(0, per_w // 2, pair, 0)
        for b in range(2):
            idx_v, line_v, rows_v, comp_v, sem, wsem = gsets[b]
            pltpu.make_async_copy(comp_v, wb_dst(t_base + per_w - 2 + b), wsem).wait()

    return body(ids_phys, w_phys)


def kernel(token_ids, W):
    nb, nh = token_ids.shape
    bt_tiles = nb // _BT
    ht_tiles = nh // _HT
    ids_phys = (
        token_ids.astype(jnp.int32)
        .reshape(bt_tiles, _BT, ht_tiles, _HT)
        .transpose(2, 0, 3, 1)
        .reshape(nb * nh)
    )
    v = W.shape[0]
    vp = (v + _VPAD - 1) // _VPAD * _VPAD
    w_phys = (
        jnp.pad(W, ((0, vp - v), (0, 0)))
        .reshape(vp // _BT, _BT, 2)
        .transpose(0, 2, 1)
        .reshape(2 * vp)
    )
    out2d, _ = _sc_gather(ids_phys, w_phys, nb, nh, vp)
    return (
        out2d.reshape(nh, bt_tiles, 2, _BT)
        .transpose(1, 3, 0, 2)
        .reshape(nb, nh, 2)
    )


# speedup vs baseline: 18.8222x; 1.0207x over previous
import functools

import jax
import jax.numpy as jnp
from jax import lax
from jax.experimental import pallas as pl
from jax.experimental.pallas import tpu as pltpu
from jax.experimental.pallas import tpu_sc as plsc

_NW = 32
_NS = 16
_L = 16
_BT = 128
_HT = 8
_VPAD = 65536
_CB = 16


@functools.partial(jax.jit, static_argnames=("nb", "nh", "vp"))
def _sc_gather(ids_phys, w_phys, nb, nh, vp):
    n_tiles = (nb // _BT) * (nh // _HT)
    per_w = n_tiles // _NW
    assert per_w % 2 == 0
    bt_tiles = nb // _BT
    row_elems = nb * 2
    tile_n = _BT * _HT

    n_blocks = vp // _BT
    chunks_per_s = n_blocks // _CB // _NS
    assert n_blocks % (_CB * _NS) == 0
    cf = _CB * 2 * _BT
    cl = _CB * _BT // 4

    mesh = plsc.VectorSubcoreMesh(core_axis_name="c", subcore_axis_name="s")

    scratch = []
    for _ in range(2):
        scratch.append(pltpu.VMEM((cf,), jnp.float32))
        scratch.append(pltpu.VMEM((cl, 8), jnp.float32))
        scratch.append(pltpu.SemaphoreType.DMA)
        scratch.append(pltpu.SemaphoreType.DMA)
    for _ in range(2):
        scratch.append(pltpu.VMEM((tile_n,), jnp.int32))
        scratch.append(pltpu.VMEM((tile_n,), jnp.int32))
        scratch.append(pltpu.VMEM((tile_n, 8), jnp.float32))
        scratch.append(pltpu.VMEM((_HT, 2 * _BT), jnp.float32))
        scratch.append(pltpu.SemaphoreType.DMA)
        scratch.append(pltpu.SemaphoreType.DMA)

    @functools.partial(
        pl.kernel,
        out_type=(
            jax.ShapeDtypeStruct((nh, row_elems), jnp.float32),
            jax.ShapeDtypeStruct((vp // 4, 8), jnp.float32),
        ),
        mesh=mesh,
        scratch_types=scratch,
        compiler_params=pltpu.CompilerParams(
            use_tc_tiling_on_sc=False, needs_layout_passes=False
        ),
    )
    def body(ids_hbm, tab_hbm, out_hbm, lines_hbm, *bufs):
        cid = lax.axis_index("c")
        sid = lax.axis_index("s")
        wid = sid * 2 + cid
        t_base = wid * per_w
        psets = tuple(tuple(bufs[4 * b : 4 * b + 4]) for b in range(2))
        gsets = tuple(tuple(bufs[8 + 6 * b : 8 + 6 * b + 6]) for b in range(2))
        iota = lax.iota(jnp.int32, _L)

        def p1_stage(ci, ib_v, isem):
            pltpu.async_copy(
                tab_hbm.at[pl.ds((sid * chunks_per_s + ci) * cf, cf)], ib_v, isem
            )

        def p1_work(ci, ib_v, pr_v, isem, wsem):
            pltpu.make_async_copy(
                tab_hbm.at[pl.ds((sid * chunks_per_s + ci) * cf, cf)], ib_v, isem
            ).wait()

            @plsc.parallel_loop(0, _CB * _BT // _L, unroll=4)
            def inter(g):
                blk = lax.shift_right_logical(g, 3)
                off = (g & 7) * _L
                src = blk * 2 * _BT + off
                x16 = ib_v[pl.ds(src, _L)]
                y16 = ib_v[pl.ds(src + _BT, _L)]
                p = blk * _BT + off + iota
                row = lax.shift_right_logical(p, 2)
                colx = lax.shift_left(p & 3, 1)
                plsc.store_scatter(pr_v, [row, colx], x16)
                plsc.store_scatter(pr_v, [row, colx + 1], y16)
            pltpu.async_copy(
                pr_v,
                lines_hbm.at[pl.ds((sid * chunks_per_s + ci) * cl, cl), :],
                wsem,
            )

        def p1_drain(ci, pr_v, wsem):
            pltpu.make_async_copy(
                pr_v,
                lines_hbm.at[pl.ds((sid * chunks_per_s + ci) * cl, cl), :],
                wsem,
            ).wait()

        for b in range(2):
            p1_stage(b, psets[b][0], psets[b][2])

        def p1_pair(k, carry):
            for b in range(2):
                ib_v, pr_v, isem, wsem = psets[b]
                ci = 2 * k + b

                @pl.when(ci >= 2)
                def _():
                    p1_drain(ci - 2, pr_v, wsem)

                p1_work(ci, ib_v, pr_v, isem, wsem)

                @pl.when(ci + 2 < chunks_per_s)
                def _():
                    p1_stage(ci + 2, ib_v, isem)

            return carry

        lax.fori_loop(0, chunks_per_s // 2, p1_pair, 0)
        for b in range(2):
            p1_drain(chunks_per_s - 2 + b, psets[b][1], psets[b][3])
        plsc.subcore_barrier()

        def stage_and_fire(t, idx_v, line_v, rows_v, sem):
            pltpu.sync_copy(ids_hbm.at[pl.ds(t * tile_n, tile_n)], idx_v)

            @plsc.parallel_loop(0, tile_n // _L, unroll=4)
            def lines(g):
                v = idx_v[pl.ds(g * _L, _L)]
                line_v[pl.ds(g * _L, _L)] = lax.shift_right_logical(v, 2)

            pltpu.async_copy(lines_hbm.at[line_v], rows_v, sem)

        def wb_dst(t):
            ht = t // bt_tiles
            bt = t - ht * bt_tiles
            return out_hbm.at[
                pl.ds(ht * _HT, _HT), pl.ds(bt * 2 * _BT, 2 * _BT)
            ]

        for b in range(2):
            idx_v, line_v, rows_v, comp_v, sem, wsem = gsets[b]
            stage_and_fire(t_base + b, idx_v, line_v, rows_v, sem)

        def pair(k, carry):
            for b in range(2):
                idx_v, line_v, rows_v, comp_v, sem, wsem = gsets[b]
                t = t_base + 2 * k + b
                pltpu.make_async_copy(lines_hbm.at[line_v], rows_v, sem).wait()

                @pl.when(2 * k + b >= 2)
                def _():
                    pltpu.make_async_copy(comp_v, wb_dst(t - 2), wsem).wait()

                for hh in range(_HT):

                    @plsc.parallel_loop(0, _BT // _L, unroll=4)
                    def extract(g, hh=hh):
                        p = hh * _BT + g * _L
                        v = idx_v[pl.ds(p, _L)]
                        col = lax.shift_left(v & 3, 1)
                        r16 = iota + p
                        x = plsc.load_gather(rows_v, [r16, col])
                        y = plsc.load_gather(rows_v, [r16, col + 1])
                        comp_v[hh, pl.ds(g * _L, _L)] = x
                        comp_v[hh, pl.ds(_BT + g * _L, _L)] = y

                @pl.when(2 * k + b + 2 < per_w)
                def _():
                    stage_and_fire(t + 2, idx_v, line_v, rows_v, sem)

                pltpu.async_copy(comp_v, wb_dst(t), wsem)
            return carry

        lax.fori_loop(0, per_w // 2, pair, 0)
        for b in range(2):
            idx_v, line_v, rows_v, comp_v, sem, wsem = gsets[b]
            pltpu.make_async_copy(comp_v, wb_dst(t_base + per_w - 2 + b), wsem).wait()

    return body(ids_phys, w_phys)


def kernel(token_ids, W):
    nb, nh = token_ids.shape
    bt_tiles = nb // _BT
    ht_tiles = nh // _HT
    ids_phys = (
        token_ids.astype(jnp.int32)
        .reshape(bt_tiles, _BT, ht_tiles, _HT)
        .transpose(2, 0, 3, 1)
        .reshape(nb * nh)
    )
    v = W.shape[0]
    vp = (v + _VPAD - 1) // _VPAD * _VPAD
    w_phys = (
        jnp.pad(W, ((0, vp - v), (0, 0)))
        .reshape(vp // _BT, _BT, 2)
        .transpose(0, 2, 1)
        .reshape(2 * vp)
    )
    out2d, _ = _sc_gather(ids_phys, w_phys, nb, nh, vp)
    return (
        out2d.reshape(nh, bt_tiles, 2, _BT)
        .transpose(1, 3, 0, 2)
        .reshape(nb, nh, 2)
    )
